# baseline (speedup 1.0000x reference)
import functools

import jax
import jax.numpy as jnp
from jax import lax
from jax.experimental import pallas as pl
from jax.experimental.pallas import tpu as pltpu
from jax.experimental.pallas import tpu_sc as plsc

VOCAB = 1000000
EMB = 32
BATCH = 16384
FIELDS = 26
TOTAL = BATCH * FIELDS

PACK = 128 // EMB
BLK = PACK * EMB
NUM_CORES = 2
NUM_SUBCORES = 16
NUM_WORKERS = NUM_CORES * NUM_SUBCORES
PER_WORKER = TOTAL // NUM_WORKERS
CHUNK = 512
NUM_CHUNKS = PER_WORKER // CHUNK
GROUPS = CHUNK // 16


def _embedding_gather(idx_flat, table_blocks):
  mesh = plsc.VectorSubcoreMesh(core_axis_name="c", subcore_axis_name="s")

  @functools.partial(
      pl.kernel,
      mesh=mesh,
      compiler_params=pltpu.CompilerParams(needs_layout_passes=False),
      out_type=jax.ShapeDtypeStruct((TOTAL * EMB,), jnp.float32),
      scratch_types=[
          pltpu.VMEM((CHUNK,), jnp.int32),
          pltpu.VMEM((CHUNK,), jnp.int32),
          pltpu.VMEM((CHUNK, BLK), jnp.float32),
          pltpu.VMEM((CHUNK * EMB,), jnp.float32),
          pltpu.SemaphoreType.DMA,
      ],
  )
  def k(idx_hbm, table_hbm, out_hbm, idx_v, bidx_v, rows_v, out_v, sem):
    wid = lax.axis_index("s") * NUM_CORES + lax.axis_index("c")
    base = wid * PER_WORKER
    lane = lax.iota(jnp.int32, 16)
    lane_dst = lane * EMB
    three = jnp.full((16,), 3, jnp.int32)

    def chunk_body(i, carry):
      off = base + i * CHUNK
      pltpu.sync_copy(idx_hbm.at[pl.ds(off, CHUNK)], idx_v)

      def blk_body(g, c):
        v = idx_v[pl.ds(g * 16, 16)]
        bidx_v[pl.ds(g * 16, 16)] = lax.shift_right_logical(v, 2)
        return c

      lax.fori_loop(0, GROUPS, blk_body, 0)
      pltpu.async_copy(table_hbm.at[bidx_v], rows_v, sem).wait()

      def sel_body(g, c):
        v = idx_v[pl.ds(g * 16, 16)]
        row = lane + g * 16
        col = (v & three) * EMB
        dst = lane_dst + g * (16 * EMB)
        for j in range(EMB):
          vals = plsc.load_gather(rows_v, [row, col + j])
          plsc.store_scatter(out_v, [dst + j], vals)
        return c

      lax.fori_loop(0, GROUPS, sel_body, 0)
      pltpu.sync_copy(out_v, out_hbm.at[pl.ds(off * EMB, CHUNK * EMB)])
      return carry

    lax.fori_loop(0, NUM_CHUNKS, chunk_body, 0)

  return k(idx_flat, table_blocks)


def kernel(env_ids, table):
  idx_flat = env_ids.astype(jnp.int32).reshape(TOTAL)
  table_blocks = table.reshape(VOCAB // PACK, BLK)
  out = _embedding_gather(idx_flat, table_blocks)
  return out.reshape(BATCH, FIELDS, EMB)


# speedup vs baseline: 1.7285x; 1.7285x over previous
import functools

import jax
import jax.numpy as jnp
from jax import lax
from jax.experimental import pallas as pl
from jax.experimental.pallas import tpu as pltpu
from jax.experimental.pallas import tpu_sc as plsc

VOCAB = 1000000
EMB = 32
BATCH = 16384
FIELDS = 26
TOTAL = BATCH * FIELDS

NUM_CORES = 2
NUM_SUBCORES = 16
NUM_WORKERS = NUM_CORES * NUM_SUBCORES
ROWS_PER_WORKER = BATCH // NUM_WORKERS
PER_WORKER = ROWS_PER_WORKER * FIELDS
CHUNK_ROWS = 32
CHUNK = CHUNK_ROWS * FIELDS
NUM_CHUNKS = ROWS_PER_WORKER // CHUNK_ROWS


def _embedding_gather(env_ids, table):
  mesh = plsc.VectorSubcoreMesh(core_axis_name="c", subcore_axis_name="s")

  @functools.partial(
      pl.kernel,
      mesh=mesh,
      compiler_params=pltpu.CompilerParams(use_tc_tiling_on_sc=False),
      out_type=jax.ShapeDtypeStruct((BATCH, FIELDS, EMB), jnp.float32),
      scratch_types=[
          pltpu.VMEM((ROWS_PER_WORKER, FIELDS), jnp.int32),
          pltpu.VMEM((2, CHUNK_ROWS, FIELDS, EMB), jnp.float32),
          pltpu.SemaphoreType.DMA,
          pltpu.SemaphoreType.DMA,
          pltpu.SemaphoreType.DMA,
          pltpu.SemaphoreType.DMA,
      ],
  )
  def k(ids_hbm, table_hbm, out_hbm, idx_v, rows_v, g0, g1, w0, w1):
    wid = lax.axis_index("s") * NUM_CORES + lax.axis_index("c")
    brow = wid * ROWS_PER_WORKER
    gsem = [g0, g1]
    wsem = [w0, w1]

    pltpu.sync_copy(ids_hbm.at[pl.ds(brow, ROWS_PER_WORKER)], idx_v)

    def gather(c):
      b = c % 2

      def row_body(i, carry):
        pltpu.async_copy(
            table_hbm.at[idx_v.at[c * CHUNK_ROWS + i]],
            rows_v.at[b].at[i],
            gsem[b],
        )
        return carry

      lax.fori_loop(0, CHUNK_ROWS, row_body, 0)
      return pltpu.make_async_copy(
          out_hbm.at[pl.ds(0, CHUNK_ROWS)], rows_v.at[b], gsem[b]
      )

    def write(c):
      return pltpu.async_copy(
          rows_v.at[c % 2],
          out_hbm.at[pl.ds(brow + c * CHUNK_ROWS, CHUNK_ROWS)],
          wsem[c % 2],
      )

    hg = {}
    hw = {}
    hg[0] = gather(0)
    for c in range(NUM_CHUNKS):
      if c + 1 < NUM_CHUNKS:
        if c + 1 >= 2:
          hw[c - 1].wait()
        hg[c + 1] = gather(c + 1)
      hg[c].wait()
      hw[c] = write(c)
    hw[NUM_CHUNKS - 2].wait()
    hw[NUM_CHUNKS - 1].wait()

  return k(env_ids, table)


def kernel(env_ids, table):
  return _embedding_gather(env_ids.astype(jnp.int32), table)
